# Initial kernel scaffold; baseline (speedup 1.0000x reference)
import jax
import jax.numpy as jnp
from jax.experimental import pallas as pl


def kernel(x, edge_index, W1, b1, W2, b2, W3, b3, Wg, a_src, a_dst, bg, Wc1, bc1, Wc2, bc2, Wr1, br1, Wr2, br2):
    raise NotImplementedError("write your pallas kernel here")


# speedup vs baseline: 16.1141x; 16.1141x over previous
import functools

import jax
import jax.numpy as jnp
from jax import lax
from jax.experimental import pallas as pl
from jax.experimental.pallas import tpu as pltpu
from jax.experimental.pallas import tpu_sc as plsc

N = 10000
E = 320000
D_IN = 128
HID = 64
HEADS = 4
OUT_DIM = 3

NC = 2
NS = 16
NW = NC * NS
EPT = E // NW
EPR = E // 8
NP = 10240
NPT = NP // NS
NPR = NP // 8

F32 = jnp.float32
I32 = jnp.int32


def _mesh():
    return plsc.VectorSubcoreMesh(
        core_axis_name="c", subcore_axis_name="s", num_cores=NC, num_subcores=NS
    )


_SC_PARAMS = pltpu.CompilerParams(use_tc_tiling_on_sc=False, needs_layout_passes=False)


def _wid(c, s):
    return s * NC + c


def _zero_rows(ref, nrows, width):
    zero = jnp.zeros((16,), F32)

    def body(i, _):
        for f in range(width // 16):
            ref[i, pl.ds(f * 16, 16)] = zero
        return 0

    lax.fori_loop(0, nrows, body, 0)


def _sc_deg(dst):
    C = 1000

    @functools.partial(
        pl.kernel,
        out_type=jax.ShapeDtypeStruct((NC, NP, 16), F32),
        mesh=_mesh(),
        compiler_params=_SC_PARAMS,
        scratch_types=[
            pltpu.VMEM((C,), I32),
            pltpu.VMEM((C, 16), F32),
            pltpu.VMEM((NPT, 16), F32),
            pltpu.VMEM_SHARED((NP, 16), F32),
        ],
    )
    def k(dst_hbm, out_hbm, didx_v, ones_v, buf_v, acc_sh):
        c = lax.axis_index("c")
        s = lax.axis_index("s")
        wid = _wid(c, s)

        one_row = jnp.where(jnp.arange(16, dtype=I32) == 0, 1.0, 0.0).astype(F32)

        def init(i, _):
            ones_v[i, :] = one_row
            return 0

        lax.fori_loop(0, C, init, 0)
        _zero_rows(buf_v, NPT, 16)
        pltpu.sync_copy(buf_v, acc_sh.at[pl.ds(s * NPT, NPT)])
        plsc.subcore_barrier()

        def body(i, _):
            base = wid * EPT + i * C
            pltpu.sync_copy(dst_hbm.at[pl.ds(base, C)], didx_v)
            pltpu.sync_copy(ones_v, acc_sh.at[didx_v], add=True)
            return 0

        lax.fori_loop(0, EPT // C, body, 0)
        plsc.subcore_barrier()
        pltpu.sync_copy(acc_sh.at[pl.ds(s * NPT, NPT)], buf_v)
        pltpu.sync_copy(buf_v, out_hbm.at[c, pl.ds(s * NPT, NPT)])

    return k(dst)


def _sc_gcn_prop(g, src, dst):
    C = 1000

    @functools.partial(
        pl.kernel,
        out_type=jax.ShapeDtypeStruct((NC, NP, HID), F32),
        mesh=_mesh(),
        compiler_params=_SC_PARAMS,
        scratch_types=[
            pltpu.VMEM((C,), I32),
            pltpu.VMEM((C,), I32),
            pltpu.VMEM((C, HID), F32),
            pltpu.VMEM_SHARED((NP, HID), F32),
            pltpu.SemaphoreType.DMA,
        ],
    )
    def k(g_hbm, src_hbm, dst_hbm, out_hbm, sidx_v, didx_v, rows_v, acc_sh, sem):
        c = lax.axis_index("c")
        s = lax.axis_index("s")
        wid = _wid(c, s)

        _zero_rows(rows_v, NPT, HID)
        pltpu.sync_copy(rows_v.at[pl.ds(0, NPT)], acc_sh.at[pl.ds(s * NPT, NPT)])
        plsc.subcore_barrier()

        def body(i, _):
            base = wid * EPT + i * C
            pltpu.sync_copy(src_hbm.at[pl.ds(base, C)], sidx_v)
            pltpu.sync_copy(dst_hbm.at[pl.ds(base, C)], didx_v)
            pltpu.async_copy(g_hbm.at[sidx_v], rows_v, sem).wait()
            pltpu.sync_copy(rows_v, acc_sh.at[didx_v], add=True)
            return 0

        lax.fori_loop(0, EPT // C, body, 0)
        plsc.subcore_barrier()
        pltpu.sync_copy(acc_sh.at[pl.ds(s * NPT, NPT)], rows_v.at[pl.ds(0, NPT)])
        pltpu.sync_copy(rows_v.at[pl.ds(0, NPT)], out_hbm.at[c, pl.ds(s * NPT, NPT)])

    return k(g, src, dst)


def _sc_logits(asad, src, dst):
    C = 400

    @functools.partial(
        pl.kernel,
        out_type=jax.ShapeDtypeStruct((HEADS, E), F32),
        mesh=_mesh(),
        compiler_params=_SC_PARAMS,
        scratch_types=[
            pltpu.VMEM((N, 8), F32),
            pltpu.VMEM((C,), I32),
            pltpu.VMEM((C,), I32),
            pltpu.VMEM((HEADS, C), F32),
        ],
    )
    def k(aa_hbm, src_hbm, dst_hbm, out_hbm, aa_v, sidx_v, didx_v, lbuf_v):
        c = lax.axis_index("c")
        s = lax.axis_index("s")
        wid = _wid(c, s)
        pltpu.sync_copy(aa_hbm, aa_v)

        def body(i, _):
            base = wid * EPT + i * C
            pltpu.sync_copy(src_hbm.at[pl.ds(base, C)], sidx_v)
            pltpu.sync_copy(dst_hbm.at[pl.ds(base, C)], didx_v)

            def group(g, _):
                sv = sidx_v[pl.ds(g * 16, 16)]
                dv = didx_v[pl.ds(g * 16, 16)]
                for h in range(HEADS):
                    a = plsc.load_gather(aa_v, [sv, jnp.full((16,), h, I32)])
                    b = plsc.load_gather(aa_v, [dv, jnp.full((16,), h + 4, I32)])
                    l = a + b
                    l = jnp.where(l >= 0.0, l, 0.2 * l)
                    lbuf_v[h, pl.ds(g * 16, 16)] = l
                return 0

            lax.fori_loop(0, C // 16, group, 0)
            for h in range(HEADS):
                pltpu.sync_copy(lbuf_v.at[h], out_hbm.at[h, pl.ds(base, C)])
            return 0

        lax.fori_loop(0, EPT // C, body, 0)

    return k(asad, src, dst)


def _sc_m_partials(logits, dst):
    C = 800

    @functools.partial(
        pl.kernel,
        out_type=jax.ShapeDtypeStruct((NW, NP), F32),
        mesh=_mesh(),
        compiler_params=_SC_PARAMS,
        scratch_types=[
            pltpu.VMEM((NP,), F32),
            pltpu.VMEM((C,), F32),
            pltpu.VMEM((C,), I32),
        ],
    )
    def k(l_hbm, dst_hbm, out_hbm, m_v, lv_v, didx_v):
        c = lax.axis_index("c")
        s = lax.axis_index("s")
        grp = jnp.where(s >= 8, 1, 0)
        head = 2 * c + grp
        rank = lax.rem(s, 8)

        neg = jnp.full((16,), -3.4e38, F32)

        def init(i, _):
            m_v[pl.ds(i * 16, 16)] = neg
            return 0

        lax.fori_loop(0, NP // 16, init, 0)

        def body(i, _):
            base = rank * EPR + i * C
            pltpu.sync_copy(dst_hbm.at[pl.ds(base, C)], didx_v)
            pltpu.sync_copy(l_hbm.at[head, pl.ds(base, C)], lv_v)

            def group(g, _):
                dv = didx_v[pl.ds(g * 16, 16)]
                l16 = lv_v[pl.ds(g * 16, 16)]

                def cond(pend):
                    return jnp.any(pend)

                def retry(pend):
                    cur = plsc.load_gather(m_v, [dv])
                    plsc.store_scatter(m_v, [dv], jnp.maximum(cur, l16), mask=pend)
                    cur2 = plsc.load_gather(m_v, [dv])
                    return pend & (cur2 < l16)

                lax.while_loop(cond, retry, jnp.ones((16,), jnp.bool_))
                return 0

            lax.fori_loop(0, C // 16, group, 0)
            return 0

        lax.fori_loop(0, EPR // C, body, 0)
        pltpu.sync_copy(m_v, out_hbm.at[head * 8 + rank])

    return k(logits, dst)


def _sc_gat_prop(hh4, mT, logits, src, dst):
    C = 400
    W = 80
    EPP = E // NS

    @functools.partial(
        pl.kernel,
        out_type=jax.ShapeDtypeStruct((HEADS * NP, W), F32),
        mesh=_mesh(),
        compiler_params=_SC_PARAMS,
        scratch_types=[
            pltpu.VMEM((C,), I32),
            pltpu.VMEM((C,), I32),
            pltpu.VMEM((C,), I32),
            pltpu.VMEM((C,), F32),
            pltpu.VMEM((N,), F32),
            pltpu.VMEM((C, HID), F32),
            pltpu.VMEM((C, W), F32),
            pltpu.VMEM_SHARED((NP, W), F32),
            pltpu.SemaphoreType.DMA,
        ],
    )
    def k(hh_hbm, m_hbm, l_hbm, src_hbm, dst_hbm, out_hbm,
          sidx_v, didx_v, gidx_v, lv_v, m_v, rows_v, out_v, acc_sh, sem):
        c = lax.axis_index("c")
        s = lax.axis_index("s")
        lanes = jnp.arange(16, dtype=I32)

        for p in range(2):
            head = 2 * c + p
            pltpu.sync_copy(m_hbm.at[head], m_v)
            _zero_rows(out_v, C, W)

            def zinit(i, _):
                pltpu.sync_copy(
                    out_v.at[pl.ds(0, 128)],
                    acc_sh.at[pl.ds(s * (NP // NS) + i * 128, 128)],
                )
                return 0

            lax.fori_loop(0, NP // NS // 128, zinit, 0)
            plsc.subcore_barrier()

            def body(i, _):
                base = s * EPP + i * C
                pltpu.sync_copy(src_hbm.at[pl.ds(base, C)], sidx_v)
                pltpu.sync_copy(dst_hbm.at[pl.ds(base, C)], didx_v)
                pltpu.sync_copy(l_hbm.at[head, pl.ds(base, C)], lv_v)

                def mkidx(g, _):
                    sv = sidx_v[pl.ds(g * 16, 16)]
                    gidx_v[pl.ds(g * 16, 16)] = sv * HEADS + head
                    return 0

                lax.fori_loop(0, C // 16, mkidx, 0)
                pltpu.async_copy(hh_hbm.at[gidx_v], rows_v, sem).wait()

                def group(g, _):
                    dv = didx_v[pl.ds(g * 16, 16)]
                    l16 = lv_v[pl.ds(g * 16, 16)]
                    mg = plsc.load_gather(m_v, [dv])
                    e16 = jnp.exp(l16 - mg)
                    elane = lanes + g * 16
                    for f in range(HID):
                        fs = jnp.full((16,), f, I32)
                        v = plsc.load_gather(rows_v, [elane, fs])
                        plsc.store_scatter(out_v, [elane, fs], v * e16)
                    plsc.store_scatter(out_v, [elane, jnp.full((16,), HID, I32)], e16)
                    return 0

                lax.fori_loop(0, C // 16, group, 0)
                pltpu.sync_copy(out_v, acc_sh.at[didx_v], add=True)
                return 0

            lax.fori_loop(0, EPP // C, body, 0)
            plsc.subcore_barrier()

            def copyout(i, _):
                r0 = s * (NP // NS) + i * 128
                pltpu.sync_copy(acc_sh.at[pl.ds(r0, 128)], out_v.at[pl.ds(0, 128)])
                pltpu.sync_copy(out_v.at[pl.ds(0, 128)], out_hbm.at[pl.ds(head * NP + r0, 128)])
                return 0

            lax.fori_loop(0, NP // NS // 128, copyout, 0)
            plsc.subcore_barrier()

    return k(hh4, mT, logits, src, dst)


_B = 2000


def _dinv_of(dg):
    deg = dg[0, :, 0] + dg[1, :, 0] + 1.0
    return lax.rsqrt(deg)


def _tc_first(x, W1, degp):
    def body(x_ref, w_ref, dg_ref, g_ref):
        dinv = _dinv_of(dg_ref[...])
        h2 = jnp.dot(x_ref[...], w_ref[...], preferred_element_type=F32)
        g_ref[...] = h2 * dinv[:, None]

    return pl.pallas_call(
        body,
        grid=(N // _B,),
        in_specs=[
            pl.BlockSpec((_B, D_IN), lambda i: (i, 0)),
            pl.BlockSpec((D_IN, HID), lambda i: (0, 0)),
            pl.BlockSpec((NC, _B, 16), lambda i: (0, i, 0)),
        ],
        out_specs=pl.BlockSpec((_B, HID), lambda i: (i, 0)),
        out_shape=jax.ShapeDtypeStruct((N, HID), F32),
    )(x, W1, degp)


def _tc_mid(acc, g, degp, b, Wn, hres):
    has_res = hres is not None

    def body(*refs):
        if has_res:
            a_ref, g_ref, dg_ref, b_ref, w_ref, r_ref, h_ref, gn_ref = refs
        else:
            a_ref, g_ref, dg_ref, b_ref, w_ref, h_ref, gn_ref = refs
        dinv = _dinv_of(dg_ref[...])
        a = a_ref[...]
        t = (a[0] + a[1] + g_ref[...]) * dinv[:, None] + b_ref[...]
        h = jnp.maximum(t, 0.0)
        if has_res:
            h = h + r_ref[...]
        h_ref[...] = h
        gn_ref[...] = (
            jnp.dot(h, w_ref[...], preferred_element_type=F32) * dinv[:, None]
        )

    in_specs = [
        pl.BlockSpec((NC, _B, HID), lambda i: (0, i, 0)),
        pl.BlockSpec((_B, HID), lambda i: (i, 0)),
        pl.BlockSpec((NC, _B, 16), lambda i: (0, i, 0)),
        pl.BlockSpec((1, HID), lambda i: (0, 0)),
        pl.BlockSpec((HID, HID), lambda i: (0, 0)),
    ]
    args = [acc, g, degp, b, Wn]
    if has_res:
        in_specs.append(pl.BlockSpec((_B, HID), lambda i: (i, 0)))
        args.append(hres)
    return pl.pallas_call(
        body,
        grid=(N // _B,),
        in_specs=in_specs,
        out_specs=[
            pl.BlockSpec((_B, HID), lambda i: (i, 0)),
            pl.BlockSpec((_B, HID), lambda i: (i, 0)),
        ],
        out_shape=[
            jax.ShapeDtypeStruct((N, HID), F32),
            jax.ShapeDtypeStruct((N, HID), F32),
        ],
    )(*args)


def _tc_gat_prep(acc, g, degp, b, hres, Wg, Asrc, Adst):
    def body(a_ref, g_ref, dg_ref, b_ref, r_ref, wg_ref, as_ref, ad_ref,
             hh_ref, aa_ref, sl_ref):
        dinv = _dinv_of(dg_ref[...])
        a = a_ref[...]
        t = (a[0] + a[1] + g_ref[...]) * dinv[:, None] + b_ref[...]
        h = jnp.maximum(t, 0.0) + r_ref[...]
        hh = jnp.dot(h, wg_ref[...], preferred_element_type=F32)
        hh_ref[...] = hh
        asv = jnp.dot(hh, as_ref[...], preferred_element_type=F32)
        adv = jnp.dot(hh, ad_ref[...], preferred_element_type=F32)
        aa_ref[...] = jnp.concatenate([asv, adv], axis=1)
        t2 = asv + adv
        sl_ref[...] = jnp.where(t2 >= 0.0, t2, 0.2 * t2)

    return pl.pallas_call(
        body,
        grid=(N // _B,),
        in_specs=[
            pl.BlockSpec((NC, _B, HID), lambda i: (0, i, 0)),
            pl.BlockSpec((_B, HID), lambda i: (i, 0)),
            pl.BlockSpec((NC, _B, 16), lambda i: (0, i, 0)),
            pl.BlockSpec((1, HID), lambda i: (0, 0)),
            pl.BlockSpec((_B, HID), lambda i: (i, 0)),
            pl.BlockSpec((HID, HEADS * HID), lambda i: (0, 0)),
            pl.BlockSpec((HEADS * HID, HEADS), lambda i: (0, 0)),
            pl.BlockSpec((HEADS * HID, HEADS), lambda i: (0, 0)),
        ],
        out_specs=[
            pl.BlockSpec((_B, HEADS * HID), lambda i: (i, 0)),
            pl.BlockSpec((_B, 2 * HEADS), lambda i: (i, 0)),
            pl.BlockSpec((_B, HEADS), lambda i: (i, 0)),
        ],
        out_shape=[
            jax.ShapeDtypeStruct((N, HEADS * HID), F32),
            jax.ShapeDtypeStruct((N, 2 * HEADS), F32),
            jax.ShapeDtypeStruct((N, HEADS), F32),
        ],
    )(acc, g, degp, b, hres, Wg, Asrc, Adst)


def _tc_m_final(mparts, sl):
    def body(mp_ref, sl_ref, m4_ref, mt_ref):
        mp = mp_ref[...]
        mh = jnp.max(mp.reshape(HEADS, 8, NP), axis=1)
        m4 = jnp.maximum(mh.T[:N, :], sl_ref[...])
        m4_ref[...] = m4
        mt_ref[...] = m4.T

    return pl.pallas_call(
        body,
        grid=(1,),
        in_specs=[
            pl.BlockSpec((NW, NP), lambda i: (0, 0)),
            pl.BlockSpec((N, HEADS), lambda i: (0, 0)),
        ],
        out_specs=[
            pl.BlockSpec((N, HEADS), lambda i: (0, 0)),
            pl.BlockSpec((HEADS, N), lambda i: (0, 0)),
        ],
        out_shape=[
            jax.ShapeDtypeStruct((N, HEADS), F32),
            jax.ShapeDtypeStruct((HEADS, N), F32),
        ],
    )(mparts, sl)


def _tc_final(accs, hh, sl, m, bg, Wc1, bc1, Wc2, bc2, Wr1, br1, Wr2, br2):
    def body(acc_ref, hh_ref, sl_ref, m_ref, bg_ref, wc1_ref, bc1_ref,
             wc2_ref, bc2_ref, wr1_ref, br1_ref, wr2_ref, br2_ref,
             cls_ref, rec_ref, hg_ref):
        acc = acc_ref[...]
        evs = jnp.exp(sl_ref[...] - m_ref[...])
        hh = hh_ref[...]
        hg = jnp.zeros((_B, HID), F32)
        for h in range(HEADS):
            num = acc[h, :, 0:HID] + evs[:, h:h + 1] * hh[:, HID * h:HID * (h + 1)]
            z = acc[h, :, HID] + evs[:, h]
            hg = hg + num / (z + 1e-16)[:, None]
        hg = hg * (1.0 / HEADS) + bg_ref[...]
        hg_ref[...] = hg
        t = jnp.maximum(jnp.dot(hg, wc1_ref[...], preferred_element_type=F32) + bc1_ref[...], 0.0)
        cls_ref[...] = jnp.dot(t, wc2_ref[...], preferred_element_type=F32) + bc2_ref[...]
        t = jnp.maximum(jnp.dot(hg, wr1_ref[...], preferred_element_type=F32) + br1_ref[...], 0.0)
        rec_ref[...] = jnp.dot(t, wr2_ref[...], preferred_element_type=F32) + br2_ref[...]

    return pl.pallas_call(
        body,
        grid=(N // _B,),
        in_specs=[
            pl.BlockSpec((HEADS, _B, 80), lambda i: (0, i, 0)),
            pl.BlockSpec((_B, HEADS * HID), lambda i: (i, 0)),
            pl.BlockSpec((_B, HEADS), lambda i: (i, 0)),
            pl.BlockSpec((_B, HEADS), lambda i: (i, 0)),
            pl.BlockSpec((1, HID), lambda i: (0, 0)),
            pl.BlockSpec((HID, HID // 2), lambda i: (0, 0)),
            pl.BlockSpec((1, HID // 2), lambda i: (0, 0)),
            pl.BlockSpec((HID // 2, OUT_DIM), lambda i: (0, 0)),
            pl.BlockSpec((1, OUT_DIM), lambda i: (0, 0)),
            pl.BlockSpec((HID, HID), lambda i: (0, 0)),
            pl.BlockSpec((1, HID), lambda i: (0, 0)),
            pl.BlockSpec((HID, D_IN), lambda i: (0, 0)),
            pl.BlockSpec((1, D_IN), lambda i: (0, 0)),
        ],
        out_specs=[
            pl.BlockSpec((_B, OUT_DIM), lambda i: (i, 0)),
            pl.BlockSpec((_B, D_IN), lambda i: (i, 0)),
            pl.BlockSpec((_B, HID), lambda i: (i, 0)),
        ],
        out_shape=[
            jax.ShapeDtypeStruct((N, OUT_DIM), F32),
            jax.ShapeDtypeStruct((N, D_IN), F32),
            jax.ShapeDtypeStruct((N, HID), F32),
        ],
    )(accs, hh, sl, m, bg, Wc1, bc1, Wc2, bc2, Wr1, br1, Wr2, br2)


def kernel(x, edge_index, W1, b1, W2, b2, W3, b3, Wg, a_src, a_dst, bg,
           Wc1, bc1, Wc2, bc2, Wr1, br1, Wr2, br2):
    src = edge_index[0]
    dst = edge_index[1]

    eye = jnp.eye(HEADS, dtype=F32)
    Asrc = (a_src[:, :, None] * eye[:, None, :]).reshape(HEADS * HID, HEADS)
    Adst = (a_dst[:, :, None] * eye[:, None, :]).reshape(HEADS * HID, HEADS)

    degp = _sc_deg(dst)
    g1 = _tc_first(x, W1, degp)
    acc1 = _sc_gcn_prop(g1, src, dst)
    h1, g2 = _tc_mid(acc1, g1, degp, b1.reshape(1, HID), W2, None)
    acc2 = _sc_gcn_prop(g2, src, dst)
    h2, g3 = _tc_mid(acc2, g2, degp, b2.reshape(1, HID), W3, h1)
    acc3 = _sc_gcn_prop(g3, src, dst)
    hh, asad, sl = _tc_gat_prep(
        acc3, g3, degp, b3.reshape(1, HID), h2, Wg, Asrc, Adst
    )
    logits = _sc_logits(asad, src, dst)
    mparts = _sc_m_partials(logits, dst)
    m4, mT = _tc_m_final(mparts, sl)
    hh4 = hh.reshape(HEADS * N, HID)
    accs = _sc_gat_prop(hh4, mT, logits, src, dst)
    accs4 = accs.reshape(HEADS, NP, 80)
    cls, rec, hg = _tc_final(
        accs4, hh, sl, m4, bg.reshape(1, HID),
        Wc1, bc1.reshape(1, HID // 2), Wc2, bc2.reshape(1, OUT_DIM),
        Wr1, br1.reshape(1, HID), Wr2, br2.reshape(1, D_IN),
    )
    return (cls, rec, hg)


# speedup vs baseline: 27.3438x; 1.6969x over previous
import functools

import jax
import jax.numpy as jnp
from jax import lax
from jax.experimental import pallas as pl
from jax.experimental.pallas import tpu as pltpu
from jax.experimental.pallas import tpu_sc as plsc

N = 10000
E = 320000
D_IN = 128
HID = 64
HEADS = 4
OUT_DIM = 3

NC = 2
NS = 16
NW = NC * NS
EPT = E // NW
EPR = E // 8
NP = 10240
NPT = NP // NS
NPR = NP // 8

F32 = jnp.float32
I32 = jnp.int32


def _mesh():
    return plsc.VectorSubcoreMesh(
        core_axis_name="c", subcore_axis_name="s", num_cores=NC, num_subcores=NS
    )


_SC_PARAMS = pltpu.CompilerParams(use_tc_tiling_on_sc=False, needs_layout_passes=False)


def _wid(c, s):
    return s * NC + c


def _zero_rows(ref, nrows, width):
    zero = jnp.zeros((16,), F32)

    def body(i, _):
        for f in range(width // 16):
            ref[i, pl.ds(f * 16, 16)] = zero
        return 0

    lax.fori_loop(0, nrows, body, 0)


def _sc_deg(dst):
    C = 1000

    @functools.partial(
        pl.kernel,
        out_type=jax.ShapeDtypeStruct((NC, NP, 16), F32),
        mesh=_mesh(),
        compiler_params=_SC_PARAMS,
        scratch_types=[
            pltpu.VMEM((C,), I32),
            pltpu.VMEM((C, 16), F32),
            pltpu.VMEM((NPT, 16), F32),
            pltpu.VMEM_SHARED((NP, 16), F32),
        ],
    )
    def k(dst_hbm, out_hbm, didx_v, ones_v, buf_v, acc_sh):
        c = lax.axis_index("c")
        s = lax.axis_index("s")
        wid = _wid(c, s)

        one_row = jnp.where(jnp.arange(16, dtype=I32) == 0, 1.0, 0.0).astype(F32)

        def init(i, _):
            ones_v[i, :] = one_row
            return 0

        lax.fori_loop(0, C, init, 0)
        _zero_rows(buf_v, NPT, 16)
        pltpu.sync_copy(buf_v, acc_sh.at[pl.ds(s * NPT, NPT)])
        plsc.subcore_barrier()

        def body(i, _):
            base = wid * EPT + i * C
            pltpu.sync_copy(dst_hbm.at[pl.ds(base, C)], didx_v)
            pltpu.sync_copy(ones_v, acc_sh.at[didx_v], add=True)
            return 0

        lax.fori_loop(0, EPT // C, body, 0)
        plsc.subcore_barrier()
        pltpu.sync_copy(acc_sh.at[pl.ds(s * NPT, NPT)], buf_v)
        pltpu.sync_copy(buf_v, out_hbm.at[c, pl.ds(s * NPT, NPT)])

    return k(dst)


def _sc_gcn_prop(g, src, dst):
    C = 1000

    @functools.partial(
        pl.kernel,
        out_type=jax.ShapeDtypeStruct((NC, NP, HID), F32),
        mesh=_mesh(),
        compiler_params=_SC_PARAMS,
        scratch_types=[
            pltpu.VMEM((C,), I32),
            pltpu.VMEM((C,), I32),
            pltpu.VMEM((C, HID), F32),
            pltpu.VMEM_SHARED((NP, HID), F32),
            pltpu.SemaphoreType.DMA,
        ],
    )
    def k(g_hbm, src_hbm, dst_hbm, out_hbm, sidx_v, didx_v, rows_v, acc_sh, sem):
        c = lax.axis_index("c")
        s = lax.axis_index("s")
        wid = _wid(c, s)

        _zero_rows(rows_v, NPT, HID)
        pltpu.sync_copy(rows_v.at[pl.ds(0, NPT)], acc_sh.at[pl.ds(s * NPT, NPT)])
        plsc.subcore_barrier()

        def body(i, _):
            base = wid * EPT + i * C
            pltpu.sync_copy(src_hbm.at[pl.ds(base, C)], sidx_v)
            pltpu.sync_copy(dst_hbm.at[pl.ds(base, C)], didx_v)
            pltpu.async_copy(g_hbm.at[sidx_v], rows_v, sem).wait()
            pltpu.sync_copy(rows_v, acc_sh.at[didx_v], add=True)
            return 0

        lax.fori_loop(0, EPT // C, body, 0)
        plsc.subcore_barrier()
        pltpu.sync_copy(acc_sh.at[pl.ds(s * NPT, NPT)], rows_v.at[pl.ds(0, NPT)])
        pltpu.sync_copy(rows_v.at[pl.ds(0, NPT)], out_hbm.at[c, pl.ds(s * NPT, NPT)])

    return k(g, src, dst)


def _sc_logits(asad, src, dst):
    C = 400

    @functools.partial(
        pl.kernel,
        out_type=jax.ShapeDtypeStruct((HEADS, E), F32),
        mesh=_mesh(),
        compiler_params=_SC_PARAMS,
        scratch_types=[
            pltpu.VMEM((N, 8), F32),
            pltpu.VMEM((C,), I32),
            pltpu.VMEM((C,), I32),
            pltpu.VMEM((HEADS, C), F32),
        ],
    )
    def k(aa_hbm, src_hbm, dst_hbm, out_hbm, aa_v, sidx_v, didx_v, lbuf_v):
        c = lax.axis_index("c")
        s = lax.axis_index("s")
        wid = _wid(c, s)
        pltpu.sync_copy(aa_hbm, aa_v)

        def body(i, _):
            base = wid * EPT + i * C
            pltpu.sync_copy(src_hbm.at[pl.ds(base, C)], sidx_v)
            pltpu.sync_copy(dst_hbm.at[pl.ds(base, C)], didx_v)

            def group(g, _):
                sv = sidx_v[pl.ds(g * 16, 16)]
                dv = didx_v[pl.ds(g * 16, 16)]
                for h in range(HEADS):
                    a = plsc.load_gather(aa_v, [sv, jnp.full((16,), h, I32)])
                    b = plsc.load_gather(aa_v, [dv, jnp.full((16,), h + 4, I32)])
                    l = a + b
                    l = jnp.where(l >= 0.0, l, 0.2 * l)
                    lbuf_v[h, pl.ds(g * 16, 16)] = l
                return 0

            lax.fori_loop(0, C // 16, group, 0)
            for h in range(HEADS):
                pltpu.sync_copy(lbuf_v.at[h], out_hbm.at[h, pl.ds(base, C)])
            return 0

        lax.fori_loop(0, EPT // C, body, 0)

    return k(asad, src, dst)


def _sc_m_partials(logits, dst):
    C = 800

    @functools.partial(
        pl.kernel,
        out_type=jax.ShapeDtypeStruct((NW, NP), F32),
        mesh=_mesh(),
        compiler_params=_SC_PARAMS,
        scratch_types=[
            pltpu.VMEM((NP,), F32),
            pltpu.VMEM((C,), F32),
            pltpu.VMEM((C,), I32),
        ],
    )
    def k(l_hbm, dst_hbm, out_hbm, m_v, lv_v, didx_v):
        c = lax.axis_index("c")
        s = lax.axis_index("s")
        grp = jnp.where(s >= 8, 1, 0)
        head = 2 * c + grp
        rank = lax.rem(s, 8)

        neg = jnp.full((16,), -3.4e38, F32)

        def init(i, _):
            m_v[pl.ds(i * 16, 16)] = neg
            return 0

        lax.fori_loop(0, NP // 16, init, 0)

        def body(i, _):
            base = rank * EPR + i * C
            pltpu.sync_copy(dst_hbm.at[pl.ds(base, C)], didx_v)
            pltpu.sync_copy(l_hbm.at[head, pl.ds(base, C)], lv_v)

            def group(g, _):
                dv = didx_v[pl.ds(g * 16, 16)]
                l16 = lv_v[pl.ds(g * 16, 16)]

                def cond(pend):
                    return jnp.any(pend)

                def retry(pend):
                    cur = plsc.load_gather(m_v, [dv])
                    plsc.store_scatter(m_v, [dv], jnp.maximum(cur, l16), mask=pend)
                    cur2 = plsc.load_gather(m_v, [dv])
                    return pend & (cur2 < l16)

                lax.while_loop(cond, retry, jnp.ones((16,), jnp.bool_))
                return 0

            lax.fori_loop(0, C // 16, group, 0)
            return 0

        lax.fori_loop(0, EPR // C, body, 0)
        pltpu.sync_copy(m_v, out_hbm.at[head * 8 + rank])

    return k(logits, dst)


def _sc_gat_prop(hh4, mT, logits, src, dst):
    C = 400
    W = 80
    EPP = E // NS

    @functools.partial(
        pl.kernel,
        out_type=jax.ShapeDtypeStruct((HEADS * NP, W), F32),
        mesh=_mesh(),
        compiler_params=_SC_PARAMS,
        scratch_types=[
            pltpu.VMEM((C,), I32),
            pltpu.VMEM((C,), I32),
            pltpu.VMEM((C,), I32),
            pltpu.VMEM((C,), F32),
            pltpu.VMEM((N,), F32),
            pltpu.VMEM((C, HID), F32),
            pltpu.VMEM((C, W), F32),
            pltpu.VMEM_SHARED((NP, W), F32),
            pltpu.SemaphoreType.DMA,
        ],
    )
    def k(hh_hbm, m_hbm, l_hbm, src_hbm, dst_hbm, out_hbm,
          sidx_v, didx_v, gidx_v, lv_v, m_v, rows_v, out_v, acc_sh, sem):
        c = lax.axis_index("c")
        s = lax.axis_index("s")
        lanes = jnp.arange(16, dtype=I32)

        for p in range(2):
            head = 2 * c + p
            pltpu.sync_copy(m_hbm.at[head], m_v)
            _zero_rows(out_v, C, W)

            def zinit(i, _):
                pltpu.sync_copy(
                    out_v.at[pl.ds(0, 128)],
                    acc_sh.at[pl.ds(s * (NP // NS) + i * 128, 128)],
                )
                return 0

            lax.fori_loop(0, NP // NS // 128, zinit, 0)
            plsc.subcore_barrier()

            def body(i, _):
                base = s * EPP + i * C
                pltpu.sync_copy(src_hbm.at[pl.ds(base, C)], sidx_v)
                pltpu.sync_copy(dst_hbm.at[pl.ds(base, C)], didx_v)
                pltpu.sync_copy(l_hbm.at[head, pl.ds(base, C)], lv_v)

                def mkidx(g, _):
                    sv = sidx_v[pl.ds(g * 16, 16)]
                    gidx_v[pl.ds(g * 16, 16)] = sv * HEADS + head
                    return 0

                lax.fori_loop(0, C // 16, mkidx, 0)
                pltpu.async_copy(hh_hbm.at[gidx_v], rows_v, sem).wait()

                def group(g, _):
                    dv = didx_v[pl.ds(g * 16, 16)]
                    l16 = lv_v[pl.ds(g * 16, 16)]
                    mg = plsc.load_gather(m_v, [dv])
                    e16 = jnp.exp(l16 - mg)
                    elane = lanes + g * 16
                    plsc.store_scatter(out_v, [elane, jnp.full((16,), HID, I32)], e16)
                    for j in range(16):
                        e = g * 16 + j
                        ev = e16[j]
                        for q in range(HID // 16):
                            out_v[e, pl.ds(q * 16, 16)] = (
                                rows_v[e, pl.ds(q * 16, 16)] * ev
                            )
                    return 0

                lax.fori_loop(0, C // 16, group, 0)
                pltpu.sync_copy(out_v, acc_sh.at[didx_v], add=True)
                return 0

            lax.fori_loop(0, EPP // C, body, 0)
            plsc.subcore_barrier()

            def copyout(i, _):
                r0 = s * (NP // NS) + i * 128
                pltpu.sync_copy(acc_sh.at[pl.ds(r0, 128)], out_v.at[pl.ds(0, 128)])
                pltpu.sync_copy(out_v.at[pl.ds(0, 128)], out_hbm.at[pl.ds(head * NP + r0, 128)])
                return 0

            lax.fori_loop(0, NP // NS // 128, copyout, 0)
            plsc.subcore_barrier()

    return k(hh4, mT, logits, src, dst)


_B = 2000


def _dinv_of(dg):
    deg = dg[0, :, 0] + dg[1, :, 0] + 1.0
    return lax.rsqrt(deg)


def _tc_first(x, W1, degp):
    def body(x_ref, w_ref, dg_ref, g_ref):
        dinv = _dinv_of(dg_ref[...])
        h2 = jnp.dot(x_ref[...], w_ref[...], preferred_element_type=F32)
        g_ref[...] = h2 * dinv[:, None]

    return pl.pallas_call(
        body,
        grid=(N // _B,),
        in_specs=[
            pl.BlockSpec((_B, D_IN), lambda i: (i, 0)),
            pl.BlockSpec((D_IN, HID), lambda i: (0, 0)),
            pl.BlockSpec((NC, _B, 16), lambda i: (0, i, 0)),
        ],
        out_specs=pl.BlockSpec((_B, HID), lambda i: (i, 0)),
        out_shape=jax.ShapeDtypeStruct((N, HID), F32),
    )(x, W1, degp)


def _tc_mid(acc, g, degp, b, Wn, hres):
    has_res = hres is not None

    def body(*refs):
        if has_res:
            a_ref, g_ref, dg_ref, b_ref, w_ref, r_ref, h_ref, gn_ref = refs
        else:
            a_ref, g_ref, dg_ref, b_ref, w_ref, h_ref, gn_ref = refs
        dinv = _dinv_of(dg_ref[...])
        a = a_ref[...]
        t = (a[0] + a[1] + g_ref[...]) * dinv[:, None] + b_ref[...]
        h = jnp.maximum(t, 0.0)
        if has_res:
            h = h + r_ref[...]
        h_ref[...] = h
        gn_ref[...] = (
            jnp.dot(h, w_ref[...], preferred_element_type=F32) * dinv[:, None]
        )

    in_specs = [
        pl.BlockSpec((NC, _B, HID), lambda i: (0, i, 0)),
        pl.BlockSpec((_B, HID), lambda i: (i, 0)),
        pl.BlockSpec((NC, _B, 16), lambda i: (0, i, 0)),
        pl.BlockSpec((1, HID), lambda i: (0, 0)),
        pl.BlockSpec((HID, HID), lambda i: (0, 0)),
    ]
    args = [acc, g, degp, b, Wn]
    if has_res:
        in_specs.append(pl.BlockSpec((_B, HID), lambda i: (i, 0)))
        args.append(hres)
    return pl.pallas_call(
        body,
        grid=(N // _B,),
        in_specs=in_specs,
        out_specs=[
            pl.BlockSpec((_B, HID), lambda i: (i, 0)),
            pl.BlockSpec((_B, HID), lambda i: (i, 0)),
        ],
        out_shape=[
            jax.ShapeDtypeStruct((N, HID), F32),
            jax.ShapeDtypeStruct((N, HID), F32),
        ],
    )(*args)


def _tc_gat_prep(acc, g, degp, b, hres, Wg, Asrc, Adst):
    def body(a_ref, g_ref, dg_ref, b_ref, r_ref, wg_ref, as_ref, ad_ref,
             hh_ref, aa_ref, sl_ref):
        dinv = _dinv_of(dg_ref[...])
        a = a_ref[...]
        t = (a[0] + a[1] + g_ref[...]) * dinv[:, None] + b_ref[...]
        h = jnp.maximum(t, 0.0) + r_ref[...]
        hh = jnp.dot(h, wg_ref[...], preferred_element_type=F32)
        hh_ref[...] = hh
        asv = jnp.dot(hh, as_ref[...], preferred_element_type=F32)
        adv = jnp.dot(hh, ad_ref[...], preferred_element_type=F32)
        aa_ref[...] = jnp.concatenate([asv, adv], axis=1)
        t2 = asv + adv
        sl_ref[...] = jnp.where(t2 >= 0.0, t2, 0.2 * t2)

    return pl.pallas_call(
        body,
        grid=(N // _B,),
        in_specs=[
            pl.BlockSpec((NC, _B, HID), lambda i: (0, i, 0)),
            pl.BlockSpec((_B, HID), lambda i: (i, 0)),
            pl.BlockSpec((NC, _B, 16), lambda i: (0, i, 0)),
            pl.BlockSpec((1, HID), lambda i: (0, 0)),
            pl.BlockSpec((_B, HID), lambda i: (i, 0)),
            pl.BlockSpec((HID, HEADS * HID), lambda i: (0, 0)),
            pl.BlockSpec((HEADS * HID, HEADS), lambda i: (0, 0)),
            pl.BlockSpec((HEADS * HID, HEADS), lambda i: (0, 0)),
        ],
        out_specs=[
            pl.BlockSpec((_B, HEADS * HID), lambda i: (i, 0)),
            pl.BlockSpec((_B, 2 * HEADS), lambda i: (i, 0)),
            pl.BlockSpec((_B, HEADS), lambda i: (i, 0)),
        ],
        out_shape=[
            jax.ShapeDtypeStruct((N, HEADS * HID), F32),
            jax.ShapeDtypeStruct((N, 2 * HEADS), F32),
            jax.ShapeDtypeStruct((N, HEADS), F32),
        ],
    )(acc, g, degp, b, hres, Wg, Asrc, Adst)


def _tc_m_final(mparts, sl):
    def body(mp_ref, sl_ref, m4_ref, mt_ref):
        mp = mp_ref[...]
        mh = jnp.max(mp.reshape(HEADS, 8, NP), axis=1)
        m4 = jnp.maximum(mh.T[:N, :], sl_ref[...])
        m4_ref[...] = m4
        mt_ref[...] = m4.T

    return pl.pallas_call(
        body,
        grid=(1,),
        in_specs=[
            pl.BlockSpec((NW, NP), lambda i: (0, 0)),
            pl.BlockSpec((N, HEADS), lambda i: (0, 0)),
        ],
        out_specs=[
            pl.BlockSpec((N, HEADS), lambda i: (0, 0)),
            pl.BlockSpec((HEADS, N), lambda i: (0, 0)),
        ],
        out_shape=[
            jax.ShapeDtypeStruct((N, HEADS), F32),
            jax.ShapeDtypeStruct((HEADS, N), F32),
        ],
    )(mparts, sl)


def _tc_final(accs, hh, sl, m, bg, Wc1, bc1, Wc2, bc2, Wr1, br1, Wr2, br2):
    def body(acc_ref, hh_ref, sl_ref, m_ref, bg_ref, wc1_ref, bc1_ref,
             wc2_ref, bc2_ref, wr1_ref, br1_ref, wr2_ref, br2_ref,
             cls_ref, rec_ref, hg_ref):
        acc = acc_ref[...]
        evs = jnp.exp(sl_ref[...] - m_ref[...])
        hh = hh_ref[...]
        hg = jnp.zeros((_B, HID), F32)
        for h in range(HEADS):
            num = acc[h, :, 0:HID] + evs[:, h:h + 1] * hh[:, HID * h:HID * (h + 1)]
            z = acc[h, :, HID] + evs[:, h]
            hg = hg + num / (z + 1e-16)[:, None]
        hg = hg * (1.0 / HEADS) + bg_ref[...]
        hg_ref[...] = hg
        t = jnp.maximum(jnp.dot(hg, wc1_ref[...], preferred_element_type=F32) + bc1_ref[...], 0.0)
        cls_ref[...] = jnp.dot(t, wc2_ref[...], preferred_element_type=F32) + bc2_ref[...]
        t = jnp.maximum(jnp.dot(hg, wr1_ref[...], preferred_element_type=F32) + br1_ref[...], 0.0)
        rec_ref[...] = jnp.dot(t, wr2_ref[...], preferred_element_type=F32) + br2_ref[...]

    return pl.pallas_call(
        body,
        grid=(N // _B,),
        in_specs=[
            pl.BlockSpec((HEADS, _B, 80), lambda i: (0, i, 0)),
            pl.BlockSpec((_B, HEADS * HID), lambda i: (i, 0)),
            pl.BlockSpec((_B, HEADS), lambda i: (i, 0)),
            pl.BlockSpec((_B, HEADS), lambda i: (i, 0)),
            pl.BlockSpec((1, HID), lambda i: (0, 0)),
            pl.BlockSpec((HID, HID // 2), lambda i: (0, 0)),
            pl.BlockSpec((1, HID // 2), lambda i: (0, 0)),
            pl.BlockSpec((HID // 2, OUT_DIM), lambda i: (0, 0)),
            pl.BlockSpec((1, OUT_DIM), lambda i: (0, 0)),
            pl.BlockSpec((HID, HID), lambda i: (0, 0)),
            pl.BlockSpec((1, HID), lambda i: (0, 0)),
            pl.BlockSpec((HID, D_IN), lambda i: (0, 0)),
            pl.BlockSpec((1, D_IN), lambda i: (0, 0)),
        ],
        out_specs=[
            pl.BlockSpec((_B, OUT_DIM), lambda i: (i, 0)),
            pl.BlockSpec((_B, D_IN), lambda i: (i, 0)),
            pl.BlockSpec((_B, HID), lambda i: (i, 0)),
        ],
        out_shape=[
            jax.ShapeDtypeStruct((N, OUT_DIM), F32),
            jax.ShapeDtypeStruct((N, D_IN), F32),
            jax.ShapeDtypeStruct((N, HID), F32),
        ],
    )(accs, hh, sl, m, bg, Wc1, bc1, Wc2, bc2, Wr1, br1, Wr2, br2)


def kernel(x, edge_index, W1, b1, W2, b2, W3, b3, Wg, a_src, a_dst, bg,
           Wc1, bc1, Wc2, bc2, Wr1, br1, Wr2, br2):
    src = edge_index[0]
    dst = edge_index[1]

    eye = jnp.eye(HEADS, dtype=F32)
    Asrc = (a_src[:, :, None] * eye[:, None, :]).reshape(HEADS * HID, HEADS)
    Adst = (a_dst[:, :, None] * eye[:, None, :]).reshape(HEADS * HID, HEADS)

    degp = _sc_deg(dst)
    g1 = _tc_first(x, W1, degp)
    acc1 = _sc_gcn_prop(g1, src, dst)
    h1, g2 = _tc_mid(acc1, g1, degp, b1.reshape(1, HID), W2, None)
    acc2 = _sc_gcn_prop(g2, src, dst)
    h2, g3 = _tc_mid(acc2, g2, degp, b2.reshape(1, HID), W3, h1)
    acc3 = _sc_gcn_prop(g3, src, dst)
    hh, asad, sl = _tc_gat_prep(
        acc3, g3, degp, b3.reshape(1, HID), h2, Wg, Asrc, Adst
    )
    logits = _sc_logits(asad, src, dst)
    mparts = _sc_m_partials(logits, dst)
    m4, mT = _tc_m_final(mparts, sl)
    hh4 = hh.reshape(HEADS * N, HID)
    accs = _sc_gat_prop(hh4, mT, logits, src, dst)
    accs4 = accs.reshape(HEADS, NP, 80)
    cls, rec, hg = _tc_final(
        accs4, hh, sl, m4, bg.reshape(1, HID),
        Wc1, bc1.reshape(1, HID // 2), Wc2, bc2.reshape(1, OUT_DIM),
        Wr1, br1.reshape(1, HID), Wr2, br2.reshape(1, D_IN),
    )
    return (cls, rec, hg)


# speedup vs baseline: 33.7990x; 1.2361x over previous
import functools

import jax
import jax.numpy as jnp
from jax import lax
from jax.experimental import pallas as pl
from jax.experimental.pallas import tpu as pltpu
from jax.experimental.pallas import tpu_sc as plsc

N = 10000
E = 320000
D_IN = 128
HID = 64
HEADS = 4
OUT_DIM = 3

NC = 2
NS = 16
NW = NC * NS
EPT = E // NW
EPR = E // 8
NP = 10240
NPT = NP // NS
NPR = NP // 8

F32 = jnp.float32
I32 = jnp.int32


def _mesh():
    return plsc.VectorSubcoreMesh(
        core_axis_name="c", subcore_axis_name="s", num_cores=NC, num_subcores=NS
    )


_SC_PARAMS = pltpu.CompilerParams(use_tc_tiling_on_sc=False, needs_layout_passes=False)


def _wid(c, s):
    return s * NC + c


def _zero_rows(ref, nrows, width):
    zero = jnp.zeros((16,), F32)

    def body(i, _):
        for f in range(width // 16):
            ref[i, pl.ds(f * 16, 16)] = zero
        return 0

    lax.fori_loop(0, nrows, body, 0)


def _sc_deg(dst):
    C = 1000

    @functools.partial(
        pl.kernel,
        out_type=jax.ShapeDtypeStruct((NC, NP, 16), F32),
        mesh=_mesh(),
        compiler_params=_SC_PARAMS,
        scratch_types=[
            pltpu.VMEM((C,), I32),
            pltpu.VMEM((C, 16), F32),
            pltpu.VMEM((NPT, 16), F32),
            pltpu.VMEM_SHARED((NP, 16), F32),
        ],
    )
    def k(dst_hbm, out_hbm, didx_v, ones_v, buf_v, acc_sh):
        c = lax.axis_index("c")
        s = lax.axis_index("s")
        wid = _wid(c, s)

        one_row = jnp.where(jnp.arange(16, dtype=I32) == 0, 1.0, 0.0).astype(F32)

        def init(i, _):
            ones_v[i, :] = one_row
            return 0

        lax.fori_loop(0, C, init, 0)
        _zero_rows(buf_v, NPT, 16)
        pltpu.sync_copy(buf_v, acc_sh.at[pl.ds(s * NPT, NPT)])
        plsc.subcore_barrier()

        def body(i, _):
            base = wid * EPT + i * C
            pltpu.sync_copy(dst_hbm.at[pl.ds(base, C)], didx_v)
            pltpu.sync_copy(ones_v, acc_sh.at[didx_v], add=True)
            return 0

        lax.fori_loop(0, EPT // C, body, 0)
        plsc.subcore_barrier()
        pltpu.sync_copy(acc_sh.at[pl.ds(s * NPT, NPT)], buf_v)
        pltpu.sync_copy(buf_v, out_hbm.at[c, pl.ds(s * NPT, NPT)])

    return k(dst)


def _sc_gcn_prop(g, src, dst):
    C = 1000

    @functools.partial(
        pl.kernel,
        out_type=jax.ShapeDtypeStruct((NC, NP, HID), F32),
        mesh=_mesh(),
        compiler_params=_SC_PARAMS,
        scratch_types=[
            pltpu.VMEM((C,), I32),
            pltpu.VMEM((C,), I32),
            pltpu.VMEM((C, HID), F32),
            pltpu.VMEM_SHARED((NP, HID), F32),
            pltpu.SemaphoreType.DMA,
        ],
    )
    def k(g_hbm, src_hbm, dst_hbm, out_hbm, sidx_v, didx_v, rows_v, acc_sh, sem):
        c = lax.axis_index("c")
        s = lax.axis_index("s")
        wid = _wid(c, s)

        _zero_rows(rows_v, NPT, HID)
        pltpu.sync_copy(rows_v.at[pl.ds(0, NPT)], acc_sh.at[pl.ds(s * NPT, NPT)])
        plsc.subcore_barrier()

        def body(i, _):
            base = wid * EPT + i * C
            pltpu.sync_copy(src_hbm.at[pl.ds(base, C)], sidx_v)
            pltpu.sync_copy(dst_hbm.at[pl.ds(base, C)], didx_v)
            pltpu.async_copy(g_hbm.at[sidx_v], rows_v, sem).wait()
            pltpu.sync_copy(rows_v, acc_sh.at[didx_v], add=True)
            return 0

        lax.fori_loop(0, EPT // C, body, 0)
        plsc.subcore_barrier()
        pltpu.sync_copy(acc_sh.at[pl.ds(s * NPT, NPT)], rows_v.at[pl.ds(0, NPT)])
        pltpu.sync_copy(rows_v.at[pl.ds(0, NPT)], out_hbm.at[c, pl.ds(s * NPT, NPT)])

    return k(g, src, dst)


def _sc_logits(asad, src, dst):
    C = 400

    @functools.partial(
        pl.kernel,
        out_type=jax.ShapeDtypeStruct((HEADS, E), F32),
        mesh=_mesh(),
        compiler_params=_SC_PARAMS,
        scratch_types=[
            pltpu.VMEM((N, 8), F32),
            pltpu.VMEM((C,), I32),
            pltpu.VMEM((C,), I32),
            pltpu.VMEM((HEADS, C), F32),
        ],
    )
    def k(aa_hbm, src_hbm, dst_hbm, out_hbm, aa_v, sidx_v, didx_v, lbuf_v):
        c = lax.axis_index("c")
        s = lax.axis_index("s")
        wid = _wid(c, s)
        pltpu.sync_copy(aa_hbm, aa_v)

        def body(i, _):
            base = wid * EPT + i * C
            pltpu.sync_copy(src_hbm.at[pl.ds(base, C)], sidx_v)
            pltpu.sync_copy(dst_hbm.at[pl.ds(base, C)], didx_v)

            def group(g, _):
                sv = sidx_v[pl.ds(g * 16, 16)]
                dv = didx_v[pl.ds(g * 16, 16)]
                for h in range(HEADS):
                    a = plsc.load_gather(aa_v, [sv, jnp.full((16,), h, I32)])
                    b = plsc.load_gather(aa_v, [dv, jnp.full((16,), h + 4, I32)])
                    l = a + b
                    l = jnp.where(l >= 0.0, l, 0.2 * l)
                    lbuf_v[h, pl.ds(g * 16, 16)] = l
                return 0

            lax.fori_loop(0, C // 16, group, 0)
            for h in range(HEADS):
                pltpu.sync_copy(lbuf_v.at[h], out_hbm.at[h, pl.ds(base, C)])
            return 0

        lax.fori_loop(0, EPT // C, body, 0)

    return k(asad, src, dst)


def _sc_m_partials(logits, dst):
    C = 4000

    @functools.partial(
        pl.kernel,
        out_type=jax.ShapeDtypeStruct((NW, NP), F32),
        mesh=_mesh(),
        compiler_params=_SC_PARAMS,
        scratch_types=[
            pltpu.VMEM((NP,), F32),
            pltpu.VMEM((C,), F32),
            pltpu.VMEM((C,), I32),
        ],
    )
    def k(l_hbm, dst_hbm, out_hbm, m_v, lv_v, didx_v):
        c = lax.axis_index("c")
        s = lax.axis_index("s")
        grp = jnp.where(s >= 8, 1, 0)
        head = 2 * c + grp
        rank = lax.rem(s, 8)

        neg = jnp.full((16,), -3.4e38, F32)

        def init(i, _):
            m_v[pl.ds(i * 16, 16)] = neg
            return 0

        lax.fori_loop(0, NP // 16, init, 0)

        def body(i, _):
            base = rank * EPR + i * C
            pltpu.sync_copy(dst_hbm.at[pl.ds(base, C)], didx_v)
            pltpu.sync_copy(l_hbm.at[head, pl.ds(base, C)], lv_v)

            def group(g, _):
                dv = didx_v[pl.ds(g * 16, 16)]
                l16 = lv_v[pl.ds(g * 16, 16)]

                def cond(pend):
                    return jnp.any(pend)

                def retry(pend):
                    cur = plsc.load_gather(m_v, [dv])
                    plsc.store_scatter(m_v, [dv], jnp.maximum(cur, l16), mask=pend)
                    cur2 = plsc.load_gather(m_v, [dv])
                    return pend & (cur2 < l16)

                lax.while_loop(cond, retry, jnp.ones((16,), jnp.bool_))
                return 0

            lax.fori_loop(0, C // 16, group, 0)
            return 0

        lax.fori_loop(0, EPR // C, body, 0)
        pltpu.sync_copy(m_v, out_hbm.at[head * 8 + rank])

    return k(logits, dst)


def _sc_gat_prop(hh80, mT, logits, src, dst):
    C = 400
    W = 80
    EPP = E // NS
    NCH = EPP // C

    @functools.partial(
        pl.kernel,
        out_type=jax.ShapeDtypeStruct((HEADS * NP, W), F32),
        mesh=_mesh(),
        compiler_params=_SC_PARAMS,
        scratch_types=[
            pltpu.VMEM((C,), I32), pltpu.VMEM((C,), I32),
            pltpu.VMEM((C,), I32), pltpu.VMEM((C,), I32),
            pltpu.VMEM((C,), I32), pltpu.VMEM((C,), I32),
            pltpu.VMEM((C,), F32), pltpu.VMEM((C,), F32),
            pltpu.VMEM((N,), F32),
            pltpu.VMEM((C, W), F32), pltpu.VMEM((C, W), F32),
            pltpu.VMEM_SHARED((NP, W), F32),
            pltpu.SemaphoreType.DMA,
            pltpu.SemaphoreType.DMA,
            pltpu.SemaphoreType.DMA,
        ],
    )
    def k(hh_hbm, m_hbm, l_hbm, src_hbm, dst_hbm, out_hbm,
          sidx0, sidx1, didx0, didx1, gidx0, gidx1, lv0, lv1,
          m_v, rows0, rows1, acc_sh, gsem0, gsem1, isem):
        c = lax.axis_index("c")
        s = lax.axis_index("s")
        lanes = jnp.arange(16, dtype=I32)
        slots = ((sidx0, didx0, gidx0, lv0, rows0, gsem0),
                 (sidx1, didx1, gidx1, lv1, rows1, gsem1))

        for p in range(2):
            head = 2 * c + p
            pltpu.sync_copy(m_hbm.at[head], m_v)
            _zero_rows(rows0, 128, W)

            def zinit(i, _):
                pltpu.sync_copy(
                    rows0.at[pl.ds(0, 128)],
                    acc_sh.at[pl.ds(s * (NP // NS) + i * 128, 128)],
                )
                return 0

            lax.fori_loop(0, NP // NS // 128, zinit, 0)
            plsc.subcore_barrier()

            def stage(cidx, slot):
                sidx_v, didx_v, gidx_v, lv_v, rows_v, gsem = slot
                base = s * EPP + cidx * C
                d1 = pltpu.async_copy(src_hbm.at[pl.ds(base, C)], sidx_v, isem)
                d2 = pltpu.async_copy(dst_hbm.at[pl.ds(base, C)], didx_v, isem)
                d3 = pltpu.async_copy(l_hbm.at[head, pl.ds(base, C)], lv_v, isem)
                d1.wait(); d2.wait(); d3.wait()

                def mkidx(g, _):
                    sv = sidx_v[pl.ds(g * 16, 16)]
                    gidx_v[pl.ds(g * 16, 16)] = sv * HEADS + head
                    return 0

                lax.fori_loop(0, C // 16, mkidx, 0)
                pltpu.async_copy(hh_hbm.at[gidx_v], rows_v, gsem)

            def consume(slot):
                sidx_v, didx_v, gidx_v, lv_v, rows_v, gsem = slot
                pltpu.make_async_copy(hh_hbm.at[gidx_v], rows_v, gsem).wait()

                def group(g, _):
                    dv = didx_v[pl.ds(g * 16, 16)]
                    l16 = lv_v[pl.ds(g * 16, 16)]
                    mg = plsc.load_gather(m_v, [dv])
                    e16 = jnp.exp(l16 - mg)
                    elane = lanes + g * 16
                    plsc.store_scatter(rows_v, [elane, jnp.full((16,), HID, I32)], e16)
                    for j in range(16):
                        e = g * 16 + j
                        ev = e16[j]
                        for q in range(HID // 16):
                            rows_v[e, pl.ds(q * 16, 16)] = (
                                rows_v[e, pl.ds(q * 16, 16)] * ev
                            )
                    return 0

                lax.fori_loop(0, C // 16, group, 0)
                pltpu.sync_copy(rows_v, acc_sh.at[didx_v], add=True)

            stage(0, slots[0])

            def pair(kk, _):
                stage(2 * kk + 1, slots[1])
                consume(slots[0])
                stage(jnp.minimum(2 * kk + 2, NCH - 1), slots[0])
                consume(slots[1])
                return 0

            lax.fori_loop(0, NCH // 2, pair, 0)
            pltpu.make_async_copy(hh_hbm.at[gidx0], rows0, gsem0).wait()
            plsc.subcore_barrier()

            def copyout(i, _):
                r0 = s * (NP // NS) + i * 128
                pltpu.sync_copy(acc_sh.at[pl.ds(r0, 128)], rows0.at[pl.ds(0, 128)])
                pltpu.sync_copy(rows0.at[pl.ds(0, 128)], out_hbm.at[pl.ds(head * NP + r0, 128)])
                return 0

            lax.fori_loop(0, NP // NS // 128, copyout, 0)
            plsc.subcore_barrier()

    return k(hh80, mT, logits, src, dst)


_B = 2000


def _dinv_of(dg):
    deg = dg[0, :, 0] + dg[1, :, 0] + 1.0
    return lax.rsqrt(deg)


def _tc_first(x, W1, degp):
    def body(x_ref, w_ref, dg_ref, g_ref):
        dinv = _dinv_of(dg_ref[...])
        h2 = jnp.dot(x_ref[...], w_ref[...], preferred_element_type=F32)
        g_ref[...] = h2 * dinv[:, None]

    return pl.pallas_call(
        body,
        grid=(N // _B,),
        in_specs=[
            pl.BlockSpec((_B, D_IN), lambda i: (i, 0)),
            pl.BlockSpec((D_IN, HID), lambda i: (0, 0)),
            pl.BlockSpec((NC, _B, 16), lambda i: (0, i, 0)),
        ],
        out_specs=pl.BlockSpec((_B, HID), lambda i: (i, 0)),
        out_shape=jax.ShapeDtypeStruct((N, HID), F32),
    )(x, W1, degp)


def _tc_mid(acc, g, degp, b, Wn, hres):
    has_res = hres is not None

    def body(*refs):
        if has_res:
            a_ref, g_ref, dg_ref, b_ref, w_ref, r_ref, h_ref, gn_ref = refs
        else:
            a_ref, g_ref, dg_ref, b_ref, w_ref, h_ref, gn_ref = refs
        dinv = _dinv_of(dg_ref[...])
        a = a_ref[...]
        t = (a[0] + a[1] + g_ref[...]) * dinv[:, None] + b_ref[...]
        h = jnp.maximum(t, 0.0)
        if has_res:
            h = h + r_ref[...]
        h_ref[...] = h
        gn_ref[...] = (
            jnp.dot(h, w_ref[...], preferred_element_type=F32) * dinv[:, None]
        )

    in_specs = [
        pl.BlockSpec((NC, _B, HID), lambda i: (0, i, 0)),
        pl.BlockSpec((_B, HID), lambda i: (i, 0)),
        pl.BlockSpec((NC, _B, 16), lambda i: (0, i, 0)),
        pl.BlockSpec((1, HID), lambda i: (0, 0)),
        pl.BlockSpec((HID, HID), lambda i: (0, 0)),
    ]
    args = [acc, g, degp, b, Wn]
    if has_res:
        in_specs.append(pl.BlockSpec((_B, HID), lambda i: (i, 0)))
        args.append(hres)
    return pl.pallas_call(
        body,
        grid=(N // _B,),
        in_specs=in_specs,
        out_specs=[
            pl.BlockSpec((_B, HID), lambda i: (i, 0)),
            pl.BlockSpec((_B, HID), lambda i: (i, 0)),
        ],
        out_shape=[
            jax.ShapeDtypeStruct((N, HID), F32),
            jax.ShapeDtypeStruct((N, HID), F32),
        ],
    )(*args)


def _tc_gat_prep(acc, g, degp, b, hres, Wg, Asrc, Adst):
    def body(a_ref, g_ref, dg_ref, b_ref, r_ref, wg_ref, as_ref, ad_ref,
             hh_ref, hh80_ref, aa_ref, sl_ref):
        dinv = _dinv_of(dg_ref[...])
        a = a_ref[...]
        t = (a[0] + a[1] + g_ref[...]) * dinv[:, None] + b_ref[...]
        h = jnp.maximum(t, 0.0) + r_ref[...]
        hh = jnp.dot(h, wg_ref[...], preferred_element_type=F32)
        hh_ref[...] = hh
        z16 = jnp.zeros((hh.shape[0], 16), F32)
        hh80_ref[...] = jnp.concatenate(
            [jnp.concatenate([hh[:, HID * q:HID * (q + 1)], z16], axis=1)
             for q in range(HEADS)], axis=1)
        asv = jnp.dot(hh, as_ref[...], preferred_element_type=F32)
        adv = jnp.dot(hh, ad_ref[...], preferred_element_type=F32)
        aa_ref[...] = jnp.concatenate([asv, adv], axis=1)
        t2 = asv + adv
        sl_ref[...] = jnp.where(t2 >= 0.0, t2, 0.2 * t2)

    return pl.pallas_call(
        body,
        grid=(N // _B,),
        in_specs=[
            pl.BlockSpec((NC, _B, HID), lambda i: (0, i, 0)),
            pl.BlockSpec((_B, HID), lambda i: (i, 0)),
            pl.BlockSpec((NC, _B, 16), lambda i: (0, i, 0)),
            pl.BlockSpec((1, HID), lambda i: (0, 0)),
            pl.BlockSpec((_B, HID), lambda i: (i, 0)),
            pl.BlockSpec((HID, HEADS * HID), lambda i: (0, 0)),
            pl.BlockSpec((HEADS * HID, HEADS), lambda i: (0, 0)),
            pl.BlockSpec((HEADS * HID, HEADS), lambda i: (0, 0)),
        ],
        out_specs=[
            pl.BlockSpec((_B, HEADS * HID), lambda i: (i, 0)),
            pl.BlockSpec((_B, HEADS * 80), lambda i: (i, 0)),
            pl.BlockSpec((_B, 2 * HEADS), lambda i: (i, 0)),
            pl.BlockSpec((_B, HEADS), lambda i: (i, 0)),
        ],
        out_shape=[
            jax.ShapeDtypeStruct((N, HEADS * HID), F32),
            jax.ShapeDtypeStruct((N, HEADS * 80), F32),
            jax.ShapeDtypeStruct((N, 2 * HEADS), F32),
            jax.ShapeDtypeStruct((N, HEADS), F32),
        ],
    )(acc, g, degp, b, hres, Wg, Asrc, Adst)


def _tc_m_final(mparts, sl):
    def body(mp_ref, sl_ref, m4_ref, mt_ref):
        mp = mp_ref[...]
        mh = jnp.max(mp.reshape(HEADS, 8, NP), axis=1)
        m4 = jnp.maximum(mh.T[:N, :], sl_ref[...])
        m4_ref[...] = m4
        mt_ref[...] = m4.T

    return pl.pallas_call(
        body,
        grid=(1,),
        in_specs=[
            pl.BlockSpec((NW, NP), lambda i: (0, 0)),
            pl.BlockSpec((N, HEADS), lambda i: (0, 0)),
        ],
        out_specs=[
            pl.BlockSpec((N, HEADS), lambda i: (0, 0)),
            pl.BlockSpec((HEADS, N), lambda i: (0, 0)),
        ],
        out_shape=[
            jax.ShapeDtypeStruct((N, HEADS), F32),
            jax.ShapeDtypeStruct((HEADS, N), F32),
        ],
    )(mparts, sl)


def _tc_final(accs, hh, sl, m, bg, Wc1, bc1, Wc2, bc2, Wr1, br1, Wr2, br2):
    def body(acc_ref, hh_ref, sl_ref, m_ref, bg_ref, wc1_ref, bc1_ref,
             wc2_ref, bc2_ref, wr1_ref, br1_ref, wr2_ref, br2_ref,
             cls_ref, rec_ref, hg_ref):
        acc = acc_ref[...]
        evs = jnp.exp(sl_ref[...] - m_ref[...])
        hh = hh_ref[...]
        hg = jnp.zeros((_B, HID), F32)
        for h in range(HEADS):
            num = acc[h, :, 0:HID] + evs[:, h:h + 1] * hh[:, HID * h:HID * (h + 1)]
            z = acc[h, :, HID] + evs[:, h]
            hg = hg + num / (z + 1e-16)[:, None]
        hg = hg * (1.0 / HEADS) + bg_ref[...]
        hg_ref[...] = hg
        t = jnp.maximum(jnp.dot(hg, wc1_ref[...], preferred_element_type=F32) + bc1_ref[...], 0.0)
        cls_ref[...] = jnp.dot(t, wc2_ref[...], preferred_element_type=F32) + bc2_ref[...]
        t = jnp.maximum(jnp.dot(hg, wr1_ref[...], preferred_element_type=F32) + br1_ref[...], 0.0)
        rec_ref[...] = jnp.dot(t, wr2_ref[...], preferred_element_type=F32) + br2_ref[...]

    return pl.pallas_call(
        body,
        grid=(N // _B,),
        in_specs=[
            pl.BlockSpec((HEADS, _B, 80), lambda i: (0, i, 0)),
            pl.BlockSpec((_B, HEADS * HID), lambda i: (i, 0)),
            pl.BlockSpec((_B, HEADS), lambda i: (i, 0)),
            pl.BlockSpec((_B, HEADS), lambda i: (i, 0)),
            pl.BlockSpec((1, HID), lambda i: (0, 0)),
            pl.BlockSpec((HID, HID // 2), lambda i: (0, 0)),
            pl.BlockSpec((1, HID // 2), lambda i: (0, 0)),
            pl.BlockSpec((HID // 2, OUT_DIM), lambda i: (0, 0)),
            pl.BlockSpec((1, OUT_DIM), lambda i: (0, 0)),
            pl.BlockSpec((HID, HID), lambda i: (0, 0)),
            pl.BlockSpec((1, HID), lambda i: (0, 0)),
            pl.BlockSpec((HID, D_IN), lambda i: (0, 0)),
            pl.BlockSpec((1, D_IN), lambda i: (0, 0)),
        ],
        out_specs=[
            pl.BlockSpec((_B, OUT_DIM), lambda i: (i, 0)),
            pl.BlockSpec((_B, D_IN), lambda i: (i, 0)),
            pl.BlockSpec((_B, HID), lambda i: (i, 0)),
        ],
        out_shape=[
            jax.ShapeDtypeStruct((N, OUT_DIM), F32),
            jax.ShapeDtypeStruct((N, D_IN), F32),
            jax.ShapeDtypeStruct((N, HID), F32),
        ],
    )(accs, hh, sl, m, bg, Wc1, bc1, Wc2, bc2, Wr1, br1, Wr2, br2)


def kernel(x, edge_index, W1, b1, W2, b2, W3, b3, Wg, a_src, a_dst, bg,
           Wc1, bc1, Wc2, bc2, Wr1, br1, Wr2, br2):
    src = edge_index[0]
    dst = edge_index[1]

    eye = jnp.eye(HEADS, dtype=F32)
    Asrc = (a_src[:, :, None] * eye[:, None, :]).reshape(HEADS * HID, HEADS)
    Adst = (a_dst[:, :, None] * eye[:, None, :]).reshape(HEADS * HID, HEADS)

    degp = _sc_deg(dst)
    g1 = _tc_first(x, W1, degp)
    acc1 = _sc_gcn_prop(g1, src, dst)
    h1, g2 = _tc_mid(acc1, g1, degp, b1.reshape(1, HID), W2, None)
    acc2 = _sc_gcn_prop(g2, src, dst)
    h2, g3 = _tc_mid(acc2, g2, degp, b2.reshape(1, HID), W3, h1)
    acc3 = _sc_gcn_prop(g3, src, dst)
    hh, hh80, asad, sl = _tc_gat_prep(
        acc3, g3, degp, b3.reshape(1, HID), h2, Wg, Asrc, Adst
    )
    logits = _sc_logits(asad, src, dst)
    mparts = _sc_m_partials(logits, dst)
    m4, mT = _tc_m_final(mparts, sl)
    hh80r = hh80.reshape(HEADS * N, 80)
    accs = _sc_gat_prop(hh80r, mT, logits, src, dst)
    accs4 = accs.reshape(HEADS, NP, 80)
    cls, rec, hg = _tc_final(
        accs4, hh, sl, m4, bg.reshape(1, HID),
        Wc1, bc1.reshape(1, HID // 2), Wc2, bc2.reshape(1, OUT_DIM),
        Wr1, br1.reshape(1, HID), Wr2, br2.reshape(1, D_IN),
    )
    return (cls, rec, hg)


# speedup vs baseline: 36.4800x; 1.0793x over previous
import functools

import jax
import jax.numpy as jnp
from jax import lax
from jax.experimental import pallas as pl
from jax.experimental.pallas import tpu as pltpu
from jax.experimental.pallas import tpu_sc as plsc

N = 10000
E = 320000
D_IN = 128
HID = 64
HEADS = 4
OUT_DIM = 3

NC = 2
NS = 16
NW = NC * NS
EPT = E // NW
EPR = E // 8
NP = 10240
NPT = NP // NS
NPR = NP // 8

F32 = jnp.float32
I32 = jnp.int32


def _mesh():
    return plsc.VectorSubcoreMesh(
        core_axis_name="c", subcore_axis_name="s", num_cores=NC, num_subcores=NS
    )


_SC_PARAMS = pltpu.CompilerParams(use_tc_tiling_on_sc=False, needs_layout_passes=False)


def _wid(c, s):
    return s * NC + c


def _zero_rows(ref, nrows, width):
    zero = jnp.zeros((16,), F32)

    def body(i, _):
        for f in range(width // 16):
            ref[i, pl.ds(f * 16, 16)] = zero
        return 0

    lax.fori_loop(0, nrows, body, 0)


def _sc_deg(dst):
    C = 1000

    @functools.partial(
        pl.kernel,
        out_type=jax.ShapeDtypeStruct((NC, NP, 16), F32),
        mesh=_mesh(),
        compiler_params=_SC_PARAMS,
        scratch_types=[
            pltpu.VMEM((C,), I32),
            pltpu.VMEM((C, 16), F32),
            pltpu.VMEM((NPT, 16), F32),
            pltpu.VMEM_SHARED((NP, 16), F32),
        ],
    )
    def k(dst_hbm, out_hbm, didx_v, ones_v, buf_v, acc_sh):
        c = lax.axis_index("c")
        s = lax.axis_index("s")
        wid = _wid(c, s)

        one_row = jnp.where(jnp.arange(16, dtype=I32) == 0, 1.0, 0.0).astype(F32)

        def init(i, _):
            ones_v[i, :] = one_row
            return 0

        lax.fori_loop(0, C, init, 0)
        _zero_rows(buf_v, NPT, 16)
        pltpu.sync_copy(buf_v, acc_sh.at[pl.ds(s * NPT, NPT)])
        plsc.subcore_barrier()

        def body(i, _):
            base = wid * EPT + i * C
            pltpu.sync_copy(dst_hbm.at[pl.ds(base, C)], didx_v)
            pltpu.sync_copy(ones_v, acc_sh.at[didx_v], add=True)
            return 0

        lax.fori_loop(0, EPT // C, body, 0)
        plsc.subcore_barrier()
        pltpu.sync_copy(acc_sh.at[pl.ds(s * NPT, NPT)], buf_v)
        pltpu.sync_copy(buf_v, out_hbm.at[c, pl.ds(s * NPT, NPT)])

    return k(dst)


def _sc_gcn_prop(g, src, dst):
    C = 200
    NCH = EPT // C

    @functools.partial(
        pl.kernel,
        out_type=jax.ShapeDtypeStruct((NC, NP, HID), F32),
        mesh=_mesh(),
        compiler_params=_SC_PARAMS,
        scratch_types=[
            pltpu.VMEM((C,), I32), pltpu.VMEM((C,), I32),
            pltpu.VMEM((C,), I32), pltpu.VMEM((C,), I32),
            pltpu.VMEM((C, HID), F32), pltpu.VMEM((C, HID), F32),
            pltpu.VMEM_SHARED((NP, HID), F32),
            pltpu.SemaphoreType.DMA,
            pltpu.SemaphoreType.DMA,
            pltpu.SemaphoreType.DMA,
            pltpu.SemaphoreType.DMA,
            pltpu.SemaphoreType.DMA,
        ],
    )
    def k(g_hbm, src_hbm, dst_hbm, out_hbm,
          sidx0, sidx1, didx0, didx1, rows0, rows1, acc_sh,
          gsem0, gsem1, ssem0, ssem1, isem):
        c = lax.axis_index("c")
        s = lax.axis_index("s")
        wid = _wid(c, s)
        slots = ((sidx0, didx0, rows0, gsem0, ssem0),
                 (sidx1, didx1, rows1, gsem1, ssem1))

        _zero_rows(rows0, 160, HID)

        def zinit(i, _):
            pltpu.sync_copy(
                rows0.at[pl.ds(0, 160)],
                acc_sh.at[pl.ds(s * NPT + i * 160, 160)],
            )
            return 0

        lax.fori_loop(0, NPT // 160, zinit, 0)
        plsc.subcore_barrier()

        def stage(cidx, slot, drain):
            sidx_v, didx_v, rows_v, gsem, ssem = slot
            if drain:
                pltpu.make_async_copy(rows_v, acc_sh.at[didx_v], ssem).wait()
            base = wid * EPT + cidx * C
            d1 = pltpu.async_copy(src_hbm.at[pl.ds(base, C)], sidx_v, isem)
            d2 = pltpu.async_copy(dst_hbm.at[pl.ds(base, C)], didx_v, isem)
            d1.wait(); d2.wait()
            pltpu.async_copy(g_hbm.at[sidx_v], rows_v, gsem)

        def consume(slot):
            sidx_v, didx_v, rows_v, gsem, ssem = slot
            pltpu.make_async_copy(g_hbm.at[sidx_v], rows_v, gsem).wait()
            pltpu.async_copy(rows_v, acc_sh.at[didx_v], ssem, add=True)

        stage(0, slots[0], drain=False)
        stage(1, slots[1], drain=False)

        def pair(kk, _):
            consume(slots[0])
            consume(slots[1])
            stage(2 * kk + 2, slots[0], drain=True)
            stage(2 * kk + 3, slots[1], drain=True)
            return 0

        lax.fori_loop(0, NCH // 2 - 1, pair, 0)
        consume(slots[0])
        consume(slots[1])
        pltpu.make_async_copy(rows0, acc_sh.at[didx0], ssem0).wait()
        pltpu.make_async_copy(rows1, acc_sh.at[didx1], ssem1).wait()
        plsc.subcore_barrier()

        def copyout(i, _):
            r0 = s * NPT + i * 160
            pltpu.sync_copy(acc_sh.at[pl.ds(r0, 160)], rows0.at[pl.ds(0, 160)])
            pltpu.sync_copy(rows0.at[pl.ds(0, 160)], out_hbm.at[c, pl.ds(r0, 160)])
            return 0

        lax.fori_loop(0, NPT // 160, copyout, 0)

    return k(g, src, dst)


def _sc_logits_max(asad, src, dst):
    C = 2000

    @functools.partial(
        pl.kernel,
        out_type=[
            jax.ShapeDtypeStruct((HEADS, E), F32),
            jax.ShapeDtypeStruct((NW, NP), F32),
        ],
        mesh=_mesh(),
        compiler_params=_SC_PARAMS,
        scratch_types=[
            pltpu.VMEM((N, 8), F32),
            pltpu.VMEM((C,), I32),
            pltpu.VMEM((C,), I32),
            pltpu.VMEM((C,), F32),
            pltpu.VMEM((NP,), F32),
            pltpu.SemaphoreType.DMA,
        ],
    )
    def k(aa_hbm, src_hbm, dst_hbm, l_hbm, mp_hbm,
          aa_v, sidx_v, didx_v, lbuf_v, m_v, isem):
        c = lax.axis_index("c")
        s = lax.axis_index("s")
        grp = jnp.where(s >= 8, 1, 0)
        head = 2 * c + grp
        rank = lax.rem(s, 8)
        pltpu.sync_copy(aa_hbm, aa_v)

        neg = jnp.full((16,), -3.4e38, F32)

        def init(i, _):
            m_v[pl.ds(i * 16, 16)] = neg
            return 0

        lax.fori_loop(0, NP // 16, init, 0)

        hs = jnp.full((16,), head, I32)
        hd = jnp.full((16,), head + 4, I32)

        def body(i, _):
            base = rank * EPR + i * C
            d1 = pltpu.async_copy(src_hbm.at[pl.ds(base, C)], sidx_v, isem)
            d2 = pltpu.async_copy(dst_hbm.at[pl.ds(base, C)], didx_v, isem)
            d1.wait(); d2.wait()

            def group(g, _):
                sv = sidx_v[pl.ds(g * 16, 16)]
                dv = didx_v[pl.ds(g * 16, 16)]
                a = plsc.load_gather(aa_v, [sv, hs])
                b = plsc.load_gather(aa_v, [dv, hd])
                l16 = a + b
                l16 = jnp.where(l16 >= 0.0, l16, 0.2 * l16)
                lbuf_v[pl.ds(g * 16, 16)] = l16

                def cond(pend):
                    return jnp.any(pend)

                def retry(pend):
                    cur = plsc.load_gather(m_v, [dv])
                    plsc.store_scatter(m_v, [dv], jnp.maximum(cur, l16), mask=pend)
                    cur2 = plsc.load_gather(m_v, [dv])
                    return pend & (cur2 < l16)

                lax.while_loop(cond, retry, jnp.ones((16,), jnp.bool_))
                return 0

            lax.fori_loop(0, C // 16, group, 0)
            pltpu.sync_copy(lbuf_v, l_hbm.at[head, pl.ds(base, C)])
            return 0

        lax.fori_loop(0, EPR // C, body, 0)
        pltpu.sync_copy(m_v, mp_hbm.at[head * 8 + rank])

    return k(asad, src, dst)


def _sc_gat_prop(hh80, mT, logits, src, dst):
    C = 400
    W = 80
    EPP = E // NS
    NCH = EPP // C

    @functools.partial(
        pl.kernel,
        out_type=jax.ShapeDtypeStruct((HEADS * NP, W), F32),
        mesh=_mesh(),
        compiler_params=_SC_PARAMS,
        scratch_types=[
            pltpu.VMEM((C,), I32), pltpu.VMEM((C,), I32),
            pltpu.VMEM((C,), I32), pltpu.VMEM((C,), I32),
            pltpu.VMEM((C,), I32), pltpu.VMEM((C,), I32),
            pltpu.VMEM((C,), F32), pltpu.VMEM((C,), F32),
            pltpu.VMEM((N,), F32),
            pltpu.VMEM((C, W), F32), pltpu.VMEM((C, W), F32),
            pltpu.VMEM_SHARED((NP, W), F32),
            pltpu.SemaphoreType.DMA,
            pltpu.SemaphoreType.DMA,
            pltpu.SemaphoreType.DMA,
            pltpu.SemaphoreType.DMA,
            pltpu.SemaphoreType.DMA,
        ],
    )
    def k(hh_hbm, m_hbm, l_hbm, src_hbm, dst_hbm, out_hbm,
          sidx0, sidx1, didx0, didx1, gidx0, gidx1, lv0, lv1,
          m_v, rows0, rows1, acc_sh, gsem0, gsem1, ssem0, ssem1, isem):
        c = lax.axis_index("c")
        s = lax.axis_index("s")
        lanes = jnp.arange(16, dtype=I32)
        slots = ((sidx0, didx0, gidx0, lv0, rows0, gsem0, ssem0),
                 (sidx1, didx1, gidx1, lv1, rows1, gsem1, ssem1))

        for p in range(2):
            head = 2 * c + p
            pltpu.sync_copy(m_hbm.at[head], m_v)
            _zero_rows(rows0, 128, W)

            def zinit(i, _):
                pltpu.sync_copy(
                    rows0.at[pl.ds(0, 128)],
                    acc_sh.at[pl.ds(s * (NP // NS) + i * 128, 128)],
                )
                return 0

            lax.fori_loop(0, NP // NS // 128, zinit, 0)
            plsc.subcore_barrier()

            def stage(cidx, slot, drain):
                sidx_v, didx_v, gidx_v, lv_v, rows_v, gsem, ssem = slot
                if drain:
                    pltpu.make_async_copy(rows_v, acc_sh.at[didx_v], ssem).wait()
                base = s * EPP + cidx * C
                d1 = pltpu.async_copy(src_hbm.at[pl.ds(base, C)], sidx_v, isem)
                d2 = pltpu.async_copy(dst_hbm.at[pl.ds(base, C)], didx_v, isem)
                d3 = pltpu.async_copy(l_hbm.at[head, pl.ds(base, C)], lv_v, isem)
                d1.wait(); d2.wait(); d3.wait()

                def mkidx(g, _):
                    sv = sidx_v[pl.ds(g * 16, 16)]
                    gidx_v[pl.ds(g * 16, 16)] = sv * HEADS + head
                    return 0

                lax.fori_loop(0, C // 16, mkidx, 0)
                pltpu.async_copy(hh_hbm.at[gidx_v], rows_v, gsem)

            def consume(slot):
                sidx_v, didx_v, gidx_v, lv_v, rows_v, gsem, ssem = slot
                pltpu.make_async_copy(hh_hbm.at[gidx_v], rows_v, gsem).wait()

                def group(g, _):
                    dv = didx_v[pl.ds(g * 16, 16)]
                    l16 = lv_v[pl.ds(g * 16, 16)]
                    mg = plsc.load_gather(m_v, [dv])
                    e16 = jnp.exp(l16 - mg)
                    elane = lanes + g * 16
                    plsc.store_scatter(rows_v, [elane, jnp.full((16,), HID, I32)], e16)
                    for j in range(16):
                        e = g * 16 + j
                        ev = e16[j]
                        for q in range(HID // 16):
                            rows_v[e, pl.ds(q * 16, 16)] = (
                                rows_v[e, pl.ds(q * 16, 16)] * ev
                            )
                    return 0

                lax.fori_loop(0, C // 16, group, 0)
                pltpu.async_copy(rows_v, acc_sh.at[didx_v], ssem, add=True)

            stage(0, slots[0], drain=False)
            stage(1, slots[1], drain=False)

            def pair(kk, _):
                consume(slots[0])
                consume(slots[1])
                stage(jnp.minimum(2 * kk + 2, NCH - 1), slots[0], drain=True)
                stage(jnp.minimum(2 * kk + 3, NCH - 1), slots[1], drain=True)
                return 0

            lax.fori_loop(0, NCH // 2 - 1, pair, 0)
            consume(slots[0])
            consume(slots[1])
            pltpu.make_async_copy(rows0, acc_sh.at[didx0], ssem0).wait()
            pltpu.make_async_copy(rows1, acc_sh.at[didx1], ssem1).wait()
            plsc.subcore_barrier()

            def copyout(i, _):
                r0 = s * (NP // NS) + i * 128
                pltpu.sync_copy(acc_sh.at[pl.ds(r0, 128)], rows0.at[pl.ds(0, 128)])
                pltpu.sync_copy(rows0.at[pl.ds(0, 128)], out_hbm.at[pl.ds(head * NP + r0, 128)])
                return 0

            lax.fori_loop(0, NP // NS // 128, copyout, 0)
            plsc.subcore_barrier()

    return k(hh80, mT, logits, src, dst)


_B = 2000


def _dinv_of(dg):
    deg = dg[0, :, 0] + dg[1, :, 0] + 1.0
    return lax.rsqrt(deg)


def _tc_first(x, W1, degp):
    def body(x_ref, w_ref, dg_ref, g_ref):
        dinv = _dinv_of(dg_ref[...])
        h2 = jnp.dot(x_ref[...], w_ref[...], preferred_element_type=F32)
        g_ref[...] = h2 * dinv[:, None]

    return pl.pallas_call(
        body,
        grid=(N // _B,),
        in_specs=[
            pl.BlockSpec((_B, D_IN), lambda i: (i, 0)),
            pl.BlockSpec((D_IN, HID), lambda i: (0, 0)),
            pl.BlockSpec((NC, _B, 16), lambda i: (0, i, 0)),
        ],
        out_specs=pl.BlockSpec((_B, HID), lambda i: (i, 0)),
        out_shape=jax.ShapeDtypeStruct((N, HID), F32),
    )(x, W1, degp)


def _tc_mid(acc, g, degp, b, Wn, hres):
    has_res = hres is not None

    def body(*refs):
        if has_res:
            a_ref, g_ref, dg_ref, b_ref, w_ref, r_ref, h_ref, gn_ref = refs
        else:
            a_ref, g_ref, dg_ref, b_ref, w_ref, h_ref, gn_ref = refs
        dinv = _dinv_of(dg_ref[...])
        a = a_ref[...]
        t = (a[0] + a[1] + g_ref[...]) * dinv[:, None] + b_ref[...]
        h = jnp.maximum(t, 0.0)
        if has_res:
            h = h + r_ref[...]
        h_ref[...] = h
        gn_ref[...] = (
            jnp.dot(h, w_ref[...], preferred_element_type=F32) * dinv[:, None]
        )

    in_specs = [
        pl.BlockSpec((NC, _B, HID), lambda i: (0, i, 0)),
        pl.BlockSpec((_B, HID), lambda i: (i, 0)),
        pl.BlockSpec((NC, _B, 16), lambda i: (0, i, 0)),
        pl.BlockSpec((1, HID), lambda i: (0, 0)),
        pl.BlockSpec((HID, HID), lambda i: (0, 0)),
    ]
    args = [acc, g, degp, b, Wn]
    if has_res:
        in_specs.append(pl.BlockSpec((_B, HID), lambda i: (i, 0)))
        args.append(hres)
    return pl.pallas_call(
        body,
        grid=(N // _B,),
        in_specs=in_specs,
        out_specs=[
            pl.BlockSpec((_B, HID), lambda i: (i, 0)),
            pl.BlockSpec((_B, HID), lambda i: (i, 0)),
        ],
        out_shape=[
            jax.ShapeDtypeStruct((N, HID), F32),
            jax.ShapeDtypeStruct((N, HID), F32),
        ],
    )(*args)


def _tc_gat_prep(acc, g, degp, b, hres, Wg, Asrc, Adst):
    def body(a_ref, g_ref, dg_ref, b_ref, r_ref, wg_ref, as_ref, ad_ref,
             hh_ref, hh80_ref, aa_ref, sl_ref):
        dinv = _dinv_of(dg_ref[...])
        a = a_ref[...]
        t = (a[0] + a[1] + g_ref[...]) * dinv[:, None] + b_ref[...]
        h = jnp.maximum(t, 0.0) + r_ref[...]
        hh = jnp.dot(h, wg_ref[...], preferred_element_type=F32)
        hh_ref[...] = hh
        z16 = jnp.zeros((hh.shape[0], 16), F32)
        hh80_ref[...] = jnp.concatenate(
            [jnp.concatenate([hh[:, HID * q:HID * (q + 1)], z16], axis=1)
             for q in range(HEADS)], axis=1)
        asv = jnp.dot(hh, as_ref[...], preferred_element_type=F32)
        adv = jnp.dot(hh, ad_ref[...], preferred_element_type=F32)
        aa_ref[...] = jnp.concatenate([asv, adv], axis=1)
        t2 = asv + adv
        sl_ref[...] = jnp.where(t2 >= 0.0, t2, 0.2 * t2)

    return pl.pallas_call(
        body,
        grid=(N // _B,),
        in_specs=[
            pl.BlockSpec((NC, _B, HID), lambda i: (0, i, 0)),
            pl.BlockSpec((_B, HID), lambda i: (i, 0)),
            pl.BlockSpec((NC, _B, 16), lambda i: (0, i, 0)),
            pl.BlockSpec((1, HID), lambda i: (0, 0)),
            pl.BlockSpec((_B, HID), lambda i: (i, 0)),
            pl.BlockSpec((HID, HEADS * HID), lambda i: (0, 0)),
            pl.BlockSpec((HEADS * HID, HEADS), lambda i: (0, 0)),
            pl.BlockSpec((HEADS * HID, HEADS), lambda i: (0, 0)),
        ],
        out_specs=[
            pl.BlockSpec((_B, HEADS * HID), lambda i: (i, 0)),
            pl.BlockSpec((_B, HEADS * 80), lambda i: (i, 0)),
            pl.BlockSpec((_B, 2 * HEADS), lambda i: (i, 0)),
            pl.BlockSpec((_B, HEADS), lambda i: (i, 0)),
        ],
        out_shape=[
            jax.ShapeDtypeStruct((N, HEADS * HID), F32),
            jax.ShapeDtypeStruct((N, HEADS * 80), F32),
            jax.ShapeDtypeStruct((N, 2 * HEADS), F32),
            jax.ShapeDtypeStruct((N, HEADS), F32),
        ],
    )(acc, g, degp, b, hres, Wg, Asrc, Adst)


def _tc_m_final(mparts, sl):
    def body(mp_ref, sl_ref, m4_ref, mt_ref):
        mp = mp_ref[...]
        mh = jnp.max(mp.reshape(HEADS, 8, NP), axis=1)
        m4 = jnp.maximum(mh.T[:N, :], sl_ref[...])
        m4_ref[...] = m4
        mt_ref[...] = m4.T

    return pl.pallas_call(
        body,
        grid=(1,),
        in_specs=[
            pl.BlockSpec((NW, NP), lambda i: (0, 0)),
            pl.BlockSpec((N, HEADS), lambda i: (0, 0)),
        ],
        out_specs=[
            pl.BlockSpec((N, HEADS), lambda i: (0, 0)),
            pl.BlockSpec((HEADS, N), lambda i: (0, 0)),
        ],
        out_shape=[
            jax.ShapeDtypeStruct((N, HEADS), F32),
            jax.ShapeDtypeStruct((HEADS, N), F32),
        ],
    )(mparts, sl)


def _tc_final(accs, hh, sl, m, bg, Wc1, bc1, Wc2, bc2, Wr1, br1, Wr2, br2):
    def body(acc_ref, hh_ref, sl_ref, m_ref, bg_ref, wc1_ref, bc1_ref,
             wc2_ref, bc2_ref, wr1_ref, br1_ref, wr2_ref, br2_ref,
             cls_ref, rec_ref, hg_ref):
        acc = acc_ref[...]
        evs = jnp.exp(sl_ref[...] - m_ref[...])
        hh = hh_ref[...]
        hg = jnp.zeros((_B, HID), F32)
        for h in range(HEADS):
            num = acc[h, :, 0:HID] + evs[:, h:h + 1] * hh[:, HID * h:HID * (h + 1)]
            z = acc[h, :, HID] + evs[:, h]
            hg = hg + num / (z + 1e-16)[:, None]
        hg = hg * (1.0 / HEADS) + bg_ref[...]
        hg_ref[...] = hg
        t = jnp.maximum(jnp.dot(hg, wc1_ref[...], preferred_element_type=F32) + bc1_ref[...], 0.0)
        cls_ref[...] = jnp.dot(t, wc2_ref[...], preferred_element_type=F32) + bc2_ref[...]
        t = jnp.maximum(jnp.dot(hg, wr1_ref[...], preferred_element_type=F32) + br1_ref[...], 0.0)
        rec_ref[...] = jnp.dot(t, wr2_ref[...], preferred_element_type=F32) + br2_ref[...]

    return pl.pallas_call(
        body,
        grid=(N // _B,),
        in_specs=[
            pl.BlockSpec((HEADS, _B, 80), lambda i: (0, i, 0)),
            pl.BlockSpec((_B, HEADS * HID), lambda i: (i, 0)),
            pl.BlockSpec((_B, HEADS), lambda i: (i, 0)),
            pl.BlockSpec((_B, HEADS), lambda i: (i, 0)),
            pl.BlockSpec((1, HID), lambda i: (0, 0)),
            pl.BlockSpec((HID, HID // 2), lambda i: (0, 0)),
            pl.BlockSpec((1, HID // 2), lambda i: (0, 0)),
            pl.BlockSpec((HID // 2, OUT_DIM), lambda i: (0, 0)),
            pl.BlockSpec((1, OUT_DIM), lambda i: (0, 0)),
            pl.BlockSpec((HID, HID), lambda i: (0, 0)),
            pl.BlockSpec((1, HID), lambda i: (0, 0)),
            pl.BlockSpec((HID, D_IN), lambda i: (0, 0)),
            pl.BlockSpec((1, D_IN), lambda i: (0, 0)),
        ],
        out_specs=[
            pl.BlockSpec((_B, OUT_DIM), lambda i: (i, 0)),
            pl.BlockSpec((_B, D_IN), lambda i: (i, 0)),
            pl.BlockSpec((_B, HID), lambda i: (i, 0)),
        ],
        out_shape=[
            jax.ShapeDtypeStruct((N, OUT_DIM), F32),
            jax.ShapeDtypeStruct((N, D_IN), F32),
            jax.ShapeDtypeStruct((N, HID), F32),
        ],
    )(accs, hh, sl, m, bg, Wc1, bc1, Wc2, bc2, Wr1, br1, Wr2, br2)


def kernel(x, edge_index, W1, b1, W2, b2, W3, b3, Wg, a_src, a_dst, bg,
           Wc1, bc1, Wc2, bc2, Wr1, br1, Wr2, br2):
    src = edge_index[0]
    dst = edge_index[1]

    eye = jnp.eye(HEADS, dtype=F32)
    Asrc = (a_src[:, :, None] * eye[:, None, :]).reshape(HEADS * HID, HEADS)
    Adst = (a_dst[:, :, None] * eye[:, None, :]).reshape(HEADS * HID, HEADS)

    degp = _sc_deg(dst)
    g1 = _tc_first(x, W1, degp)
    acc1 = _sc_gcn_prop(g1, src, dst)
    h1, g2 = _tc_mid(acc1, g1, degp, b1.reshape(1, HID), W2, None)
    acc2 = _sc_gcn_prop(g2, src, dst)
    h2, g3 = _tc_mid(acc2, g2, degp, b2.reshape(1, HID), W3, h1)
    acc3 = _sc_gcn_prop(g3, src, dst)
    hh, hh80, asad, sl = _tc_gat_prep(
        acc3, g3, degp, b3.reshape(1, HID), h2, Wg, Asrc, Adst
    )
    logits, mparts = _sc_logits_max(asad, src, dst)
    m4, mT = _tc_m_final(mparts, sl)
    hh80r = hh80.reshape(HEADS * N, 80)
    accs = _sc_gat_prop(hh80r, mT, logits, src, dst)
    accs4 = accs.reshape(HEADS, NP, 80)
    cls, rec, hg = _tc_final(
        accs4, hh, sl, m4, bg.reshape(1, HID),
        Wc1, bc1.reshape(1, HID // 2), Wc2, bc2.reshape(1, OUT_DIM),
        Wr1, br1.reshape(1, HID), Wr2, br2.reshape(1, D_IN),
    )
    return (cls, rec, hg)
